# baseline (speedup 1.0000x reference)
import functools

import jax
import jax.numpy as jnp
from jax import lax
from jax.experimental import pallas as pl
from jax.experimental.pallas import tpu as pltpu
from jax.experimental.pallas import tpu_sc as plsc

HEADS = 16
SEQ = 1025
NREL = 3972
ROW_LEN = 1040
NW = 32
UPW = 16
GROUPS = ROW_LEN // 16
IDX_ROWS = 1028


@functools.partial(
    pl.kernel,
    out_type=jax.ShapeDtypeStruct((SEQ, HEADS, SEQ), jnp.float32),
    mesh=plsc.VectorSubcoreMesh(core_axis_name="c", subcore_axis_name="s"),
    compiler_params=pltpu.CompilerParams(needs_layout_passes=False),
    scratch_types=[
        pltpu.VMEM((8 * NREL,), jnp.float32),
        pltpu.VMEM((4 * ROW_LEN,), jnp.int32),
        pltpu.VMEM((4 * ROW_LEN,), jnp.int32),
        pltpu.VMEM((4, 8, SEQ), jnp.float32),
        pltpu.VMEM((4, 8, SEQ), jnp.float32),
        pltpu.SemaphoreType.DMA,
        pltpu.SemaphoreType.DMA,
        pltpu.SemaphoreType.DMA,
        pltpu.SemaphoreType.DMA,
    ],
)
def _rpb_sc(
    table_hbm,
    idx_hbm,
    out_hbm,
    table_v,
    idx0,
    idx1,
    ob0,
    ob1,
    sem_i0,
    sem_i1,
    sem_o0,
    sem_o1,
):
    cid = lax.axis_index("c")
    sid = lax.axis_index("s")
    wid = sid * 2 + cid

    hg = wid % 2
    h0 = pl.multiple_of(hg * 8, 8)
    rq0 = wid // 2
    pltpu.sync_copy(table_hbm.at[pl.ds(hg * (8 * NREL), 8 * NREL)], table_v)

    lane = lax.iota(jnp.int32, 16)
    last_col = jnp.full((16,), SEQ - 1, jnp.int32)
    last_mask = lane < 1

    def idx_src(t):
        rq = rq0 + 16 * t
        return idx_hbm.at[pl.ds(rq * (4 * ROW_LEN), 4 * ROW_LEN)]

    def out_dst(t, nrows=4):
        rq = rq0 + 16 * t
        return out_hbm.at[pl.ds(rq * 4, nrows), pl.ds(h0, 8), :]

    def compute(idx_v, ob, nrows):
        for row in range(nrows):

            @plsc.parallel_loop(0, GROUPS - 1, unroll=16)
            def _g(c):
                iv = idx_v[pl.ds(row * ROW_LEN + c * 16, 16)]
                for h in range(8):
                    ob[row, h, pl.ds(c * 16, 16)] = plsc.load_gather(
                        table_v, [iv + h * NREL]
                    )

            ivl = idx_v[pl.ds(row * ROW_LEN + (SEQ - 1), 16)]
            row_idx = jnp.full((16,), row, jnp.int32)
            for h in range(8):
                vals = plsc.load_gather(table_v, [ivl + h * NREL])
                plsc.store_scatter(
                    ob,
                    [row_idx, jnp.full((16,), h, jnp.int32), last_col],
                    vals,
                    mask=last_mask,
                )

    def body(t, b, idx_b, idx_n, ob_b, sem_i_b, sem_i_n, sem_o_b):
        @pl.when(t + 1 < UPW)
        def _():
            pltpu.async_copy(idx_src(t + 1), idx_n, sem_i_n)

        pltpu.make_async_copy(idx_src(t), idx_b, sem_i_b).wait()

        @pl.when(t >= 2)
        def _():
            pltpu.make_async_copy(ob_b, out_dst(t - 2), sem_o_b).wait()

        compute(idx_b, ob_b, 4)
        pltpu.async_copy(ob_b, out_dst(t), sem_o_b)

    pltpu.async_copy(idx_src(0), idx0, sem_i0)

    def pair(k, carry):
        body(2 * k, 0, idx0, idx1, ob0, sem_i0, sem_i1, sem_o0)
        body(2 * k + 1, 1, idx1, idx0, ob1, sem_i1, sem_i0, sem_o1)
        return carry

    lax.fori_loop(0, UPW // 2, pair, 0, unroll=False)

    pltpu.make_async_copy(ob0, out_dst(UPW - 2), sem_o0).wait()
    pltpu.make_async_copy(ob1, out_dst(UPW - 1), sem_o1).wait()

    @pl.when(wid < 2)
    def _():
        pltpu.sync_copy(
            idx_hbm.at[pl.ds(1024 * ROW_LEN, ROW_LEN)],
            idx0.at[pl.ds(0, ROW_LEN)],
        )
        compute(idx0, ob0, 1)
        pltpu.sync_copy(
            ob0.at[pl.ds(0, 1), :, :],
            out_hbm.at[pl.ds(1024, 1), pl.ds(h0, 8), :],
        )


def kernel(relative_position_bias_table, relative_position_index):
    flat_t = relative_position_bias_table.T.reshape(-1)
    idx_p = (
        jnp.zeros((IDX_ROWS, ROW_LEN), jnp.int32)
        .at[:SEQ, :SEQ]
        .set(relative_position_index)
        .reshape(-1)
    )
    out = _rpb_sc(flat_t, idx_p)
    return jnp.transpose(out, (1, 0, 2))


# speedup vs baseline: 1.4458x; 1.4458x over previous
import functools

import jax
import jax.numpy as jnp
from jax import lax
from jax.experimental import pallas as pl
from jax.experimental.pallas import tpu as pltpu
from jax.experimental.pallas import tpu_sc as plsc

HEADS = 16
SEQ = 1025
NREL = 3972
ROW_LEN = 1040
NW = 32
UPW = 16
GROUPS = ROW_LEN // 16
IDX_ROWS = 1028


@functools.partial(
    pl.kernel,
    out_type=jax.ShapeDtypeStruct((SEQ, HEADS, SEQ), jnp.float32),
    mesh=plsc.VectorSubcoreMesh(core_axis_name="c", subcore_axis_name="s"),
    compiler_params=pltpu.CompilerParams(needs_layout_passes=False),
    scratch_types=[
        pltpu.VMEM((8 * NREL,), jnp.float32),
        pltpu.VMEM((4 * ROW_LEN,), jnp.int32),
        pltpu.VMEM((4 * ROW_LEN,), jnp.int32),
        pltpu.VMEM((4, 8, SEQ), jnp.float32),
        pltpu.VMEM((4, 8, SEQ), jnp.float32),
        pltpu.SemaphoreType.DMA,
        pltpu.SemaphoreType.DMA,
        pltpu.SemaphoreType.DMA,
        pltpu.SemaphoreType.DMA,
    ],
)
def _rpb_sc(
    table_hbm,
    idx_hbm,
    out_hbm,
    table_v,
    idx0,
    idx1,
    ob0,
    ob1,
    sem_i0,
    sem_i1,
    sem_o0,
    sem_o1,
):
    cid = lax.axis_index("c")
    sid = lax.axis_index("s")
    wid = sid * 2 + cid

    hg = wid % 2
    h0 = pl.multiple_of(hg * 8, 8)
    rq0 = wid // 2
    pltpu.sync_copy(table_hbm.at[pl.ds(hg * (8 * NREL), 8 * NREL)], table_v)

    lane = lax.iota(jnp.int32, 16)
    last_col = jnp.full((16,), SEQ - 1, jnp.int32)
    last_mask = lane < 1

    def idx_src(t):
        rq = rq0 + 16 * t
        return idx_hbm.at[pl.ds(rq * (4 * ROW_LEN), 4 * ROW_LEN)]

    def out_dst(t, nrows=4):
        rq = rq0 + 16 * t
        return out_hbm.at[pl.ds(rq * 4, nrows), pl.ds(h0, 8), :]

    def compute(idx_v, ob, nrows):
        for row in range(nrows):

            @plsc.parallel_loop(0, GROUPS - 1, unroll=8)
            def _g(c):
                iv = idx_v[pl.ds(row * ROW_LEN + c * 16, 16)]
                for h in range(8):
                    ob[row, h, pl.ds(c * 16, 16)] = plsc.load_gather(
                        table_v, [iv + h * NREL]
                    )

            ivl = idx_v[pl.ds(row * ROW_LEN + (SEQ - 1), 16)]
            row_idx = jnp.full((16,), row, jnp.int32)
            for h in range(8):
                vals = plsc.load_gather(table_v, [ivl + h * NREL])
                plsc.store_scatter(
                    ob,
                    [row_idx, jnp.full((16,), h, jnp.int32), last_col],
                    vals,
                    mask=last_mask,
                )

    def body(t, b, idx_b, idx_n, ob_b, sem_i_b, sem_i_n, sem_o_b):
        @pl.when(t + 1 < UPW)
        def _():
            pltpu.async_copy(idx_src(t + 1), idx_n, sem_i_n)

        pltpu.make_async_copy(idx_src(t), idx_b, sem_i_b).wait()

        @pl.when(t >= 2)
        def _():
            pltpu.make_async_copy(ob_b, out_dst(t - 2), sem_o_b).wait()

        compute(idx_b, ob_b, 4)
        pltpu.async_copy(ob_b, out_dst(t), sem_o_b)

    pltpu.async_copy(idx_src(0), idx0, sem_i0)

    def pair(k, carry):
        body(2 * k, 0, idx0, idx1, ob0, sem_i0, sem_i1, sem_o0)
        body(2 * k + 1, 1, idx1, idx0, ob1, sem_i1, sem_i0, sem_o1)
        return carry

    lax.fori_loop(0, UPW // 2, pair, 0, unroll=False)

    pltpu.make_async_copy(ob0, out_dst(UPW - 2), sem_o0).wait()
    pltpu.make_async_copy(ob1, out_dst(UPW - 1), sem_o1).wait()

    @pl.when(wid < 2)
    def _():
        pltpu.sync_copy(
            idx_hbm.at[pl.ds(1024 * ROW_LEN, ROW_LEN)],
            idx0.at[pl.ds(0, ROW_LEN)],
        )
        compute(idx0, ob0, 1)
        pltpu.sync_copy(
            ob0.at[pl.ds(0, 1), :, :],
            out_hbm.at[pl.ds(1024, 1), pl.ds(h0, 8), :],
        )


def kernel(relative_position_bias_table, relative_position_index):
    flat_t = relative_position_bias_table.T.reshape(-1)
    idx_p = (
        jnp.zeros((IDX_ROWS, ROW_LEN), jnp.int32)
        .at[:SEQ, :SEQ]
        .set(relative_position_index)
        .reshape(-1)
    )
    out = _rpb_sc(flat_t, idx_p)
    return jnp.transpose(out, (1, 0, 2))
